# Initial kernel scaffold; baseline (speedup 1.0000x reference)
import jax
import jax.numpy as jnp
from jax.experimental import pallas as pl


def kernel(node_ids, edge_index, batch, emb, W1, a_src1, a_dst1, b1, W2, a_src2, a_dst2, b2, W3, a_src3, a_dst3, b3):
    raise NotImplementedError("write your pallas kernel here")


# speedup vs baseline: 11.6351x; 11.6351x over previous
import functools

import jax
import jax.numpy as jnp
from jax import lax
from jax.experimental import pallas as pl
from jax.experimental.pallas import tpu as pltpu
from jax.experimental.pallas import tpu_sc as plsc

N = 10000
NP = 10240
E = 320000
D = 128
G = 64
NW = 32
K = 64
NCH = 157
EP = NW * K * NCH
EPW = EP // NW
RPW = NP // NW
ZR = NP // 16
DR = NP // D

_mesh = plsc.VectorSubcoreMesh(core_axis_name="c", subcore_axis_name="s")
_f32 = jnp.float32
_i32 = jnp.int32
_sc_params = pltpu.CompilerParams(needs_layout_passes=False)


@functools.partial(
    pl.kernel,
    out_type=jax.ShapeDtypeStruct((NP, D), _f32),
    mesh=_mesh,
    scratch_types=[
        pltpu.VMEM((K,), _i32),
        pltpu.VMEM((K, D), _f32),
        pltpu.SemaphoreType.DMA,
    ],
    compiler_params=_sc_params,
)
def _emb_gather(ids_hbm, emb_hbm, x_hbm, idx_v, rows_v, sem):
    c = lax.axis_index("c")
    s = lax.axis_index("s")
    wid = s * 2 + c
    base = wid * RPW
    for ch in range(RPW // K):
        off = base + ch * K
        pltpu.sync_copy(ids_hbm.at[pl.ds(off, K)], idx_v)
        pltpu.async_copy(emb_hbm.at[idx_v], rows_v, sem).wait()
        pltpu.sync_copy(rows_v, x_hbm.at[pl.ds(off, K)])


@functools.partial(
    pl.kernel,
    out_type=(
        jax.ShapeDtypeStruct((2, NP, D), _f32),
        jax.ShapeDtypeStruct((2, DR, D), _f32),
    ),
    mesh=_mesh,
    scratch_types=[
        pltpu.VMEM((NP,), _f32),
        pltpu.VMEM((NP,), _f32),
        pltpu.VMEM((K,), _i32),
        pltpu.VMEM((K,), _i32),
        pltpu.VMEM((K,), _f32),
        pltpu.VMEM((DR,), _i32),
        pltpu.VMEM((DR, D), _f32),
        pltpu.VMEM((K, D), _f32),
        pltpu.VMEM((K, D), _f32),
        pltpu.VMEM_SHARED((NP, D), _f32),
        pltpu.VMEM_SHARED((DR, D), _f32),
        pltpu.SemaphoreType.DMA,
    ],
    compiler_params=_sc_params,
)
def _edge_accum(h_hbm, as_hbm, ad_hbm, src_hbm, dst_hbm, feat_hbm, den_hbm,
                asv, adv, srcv, dstv, exv, idxr, denv, rows, rowsa,
                feat_sh, den_sh, sem):
    c = lax.axis_index("c")
    s = lax.axis_index("s")
    wid = s * 2 + c
    zero16 = jnp.zeros((16,), _f32)
    iota16 = lax.iota(_i32, 16)

    def _zb(i, _):
        r = i // (D // 16)
        col = (i % (D // 16)) * 16
        rowsa[r, pl.ds(col, 16)] = zero16
        return 0
    lax.fori_loop(0, K * D // 16, _zb, 0)

    def _zd(i, _):
        r = i // (D // 16)
        col = (i % (D // 16)) * 16
        denv[r, pl.ds(col, 16)] = zero16
        return 0
    lax.fori_loop(0, DR * D // 16, _zd, 0)
    for g in range(DR // 16):
        idxr[pl.ds(g * 16, 16)] = iota16 + g * 16

    zbase = s * ZR

    def _zsh(i, _):
        pltpu.sync_copy(rowsa, feat_sh.at[pl.ds(zbase + i * K, K)])
        return 0
    lax.fori_loop(0, ZR // K, _zsh, 0)

    @pl.when(s == 0)
    def _():
        pltpu.sync_copy(denv, den_sh)

    pltpu.sync_copy(as_hbm, asv)
    pltpu.sync_copy(ad_hbm, adv)
    plsc.subcore_barrier()

    ebase = wid * EPW

    def _chunk(ci, _):
        off = ebase + ci * K
        pltpu.sync_copy(src_hbm.at[pl.ds(off, K)], srcv)
        pltpu.sync_copy(dst_hbm.at[pl.ds(off, K)], dstv)
        pltpu.async_copy(h_hbm.at[srcv], rows, sem).wait()
        for g in range(K // 16):
            si = srcv[pl.ds(g * 16, 16)]
            di = dstv[pl.ds(g * 16, 16)]
            e = plsc.load_gather(asv, [si]) + plsc.load_gather(adv, [di])
            e = jnp.where(e >= 0, e, 0.2 * e)
            ex = jnp.exp(e)
            exv[pl.ds(g * 16, 16)] = ex
            plsc.addupdate_scatter(
                denv,
                [lax.shift_right_logical(di, 7), lax.bitwise_and(di, 127)],
                ex)

        def _scale(i, _):
            exi = plsc.load_gather(exv, [jnp.zeros((16,), _i32) + i])
            for j in range(D // 16):
                rowsa[i, pl.ds(j * 16, 16)] = rows[i, pl.ds(j * 16, 16)] * exi
            return 0
        lax.fori_loop(0, K, _scale, 0)
        pltpu.sync_copy(rowsa, feat_sh.at[dstv], add=True)
        return 0
    lax.fori_loop(0, NCH, _chunk, 0)

    plsc.subcore_barrier()
    pltpu.sync_copy(denv, den_sh.at[idxr], add=True)
    plsc.subcore_barrier()

    def _wout(i, _):
        pltpu.sync_copy(feat_sh.at[pl.ds(zbase + i * K, K)], rowsa)
        pltpu.sync_copy(rowsa, feat_hbm.at[c, pl.ds(zbase + i * K, K)])
        return 0
    lax.fori_loop(0, ZR // K, _wout, 0)

    @pl.when(s == 0)
    def _():
        pltpu.sync_copy(den_sh, rows.at[pl.ds(0, DR)])
        pltpu.sync_copy(rows.at[pl.ds(0, DR)], den_hbm.at[c])


_BLK = 1024
_NBLK = NP // _BLK


def _alphas(h, asr, adr):
    a_s = (h * asr).sum(-1, keepdims=True)
    a_d = (h * adr).sum(-1, keepdims=True)
    return a_s, a_d


def _mm_alpha_body(x_ref, w_ref, asr_ref, adr_ref, h_ref, als_ref, ald_ref):
    h = jnp.dot(x_ref[...], w_ref[...], preferred_element_type=_f32)
    h_ref[...] = h
    a_s, a_d = _alphas(h, asr_ref[...], adr_ref[...])
    als_ref[...] = a_s
    ald_ref[...] = a_d


def _combine(feat, den, h, a_s, a_d, b):
    exs = jnp.exp(jnp.where(a_s + a_d >= 0, a_s + a_d, 0.2 * (a_s + a_d)))
    num = feat[0] + feat[1] + exs * h
    dn = den[0] + den[1] + exs + 1e-16
    return num / dn + b


def _comb_mm_body(feat_ref, den_ref, h_ref, als_ref, ald_ref, b_ref,
                  w_ref, asr_ref, adr_ref, hn_ref, alsn_ref, aldn_ref):
    y = _combine(feat_ref[...], den_ref[...], h_ref[...], als_ref[...],
                 ald_ref[...], b_ref[...])
    y = jnp.maximum(y, 0.0)
    hn = jnp.dot(y, w_ref[...], preferred_element_type=_f32)
    hn_ref[...] = hn
    a_s, a_d = _alphas(hn, asr_ref[...], adr_ref[...])
    alsn_ref[...] = a_s
    aldn_ref[...] = a_d


def _pool_body(feat_ref, den_ref, h_ref, als_ref, ald_ref, b_ref, batch_ref,
               out_ref, acc, cnt):
    i = pl.program_id(0)

    @pl.when(i == 0)
    def _():
        acc[...] = jnp.zeros((G, D), _f32)
        cnt[...] = jnp.zeros((G, D), _f32)

    y = _combine(feat_ref[...], den_ref[...], h_ref[...], als_ref[...],
                 ald_ref[...], b_ref[...])
    oh = (batch_ref[...] == lax.broadcasted_iota(_i32, (1, G), 1)).astype(_f32)
    dn = (((0,), (0,)), ((), ()))
    acc[...] += lax.dot_general(oh, y, dn, preferred_element_type=_f32)
    cnt[...] += lax.dot_general(oh, jnp.ones_like(y), dn, preferred_element_type=_f32)

    @pl.when(i == _NBLK - 1)
    def _():
        out_ref[...] = acc[...] / jnp.maximum(cnt[...], 1.0)


_row_spec = pl.BlockSpec((_BLK, D), lambda i: (i, 0))
_col_spec = pl.BlockSpec((_BLK, 1), lambda i: (i, 0))
_w_spec = pl.BlockSpec((D, D), lambda i: (0, 0))
_v_spec = pl.BlockSpec((1, D), lambda i: (0, 0))
_feat_spec = pl.BlockSpec((2, _BLK, D), lambda i: (0, i, 0))
_den_spec = pl.BlockSpec((2, _BLK, 1), lambda i: (0, i, 0))

_mm_alpha = pl.pallas_call(
    _mm_alpha_body,
    grid=(_NBLK,),
    in_specs=[_row_spec, _w_spec, _v_spec, _v_spec],
    out_specs=[_row_spec, _col_spec, _col_spec],
    out_shape=[
        jax.ShapeDtypeStruct((NP, D), _f32),
        jax.ShapeDtypeStruct((NP, 1), _f32),
        jax.ShapeDtypeStruct((NP, 1), _f32),
    ],
)

_comb_mm = pl.pallas_call(
    _comb_mm_body,
    grid=(_NBLK,),
    in_specs=[_feat_spec, _den_spec, _row_spec, _col_spec, _col_spec, _v_spec,
              _w_spec, _v_spec, _v_spec],
    out_specs=[_row_spec, _col_spec, _col_spec],
    out_shape=[
        jax.ShapeDtypeStruct((NP, D), _f32),
        jax.ShapeDtypeStruct((NP, 1), _f32),
        jax.ShapeDtypeStruct((NP, 1), _f32),
    ],
)

_pool = pl.pallas_call(
    _pool_body,
    grid=(_NBLK,),
    in_specs=[_feat_spec, _den_spec, _row_spec, _col_spec, _col_spec, _v_spec,
              pl.BlockSpec((_BLK, 1), lambda i: (i, 0))],
    out_specs=pl.BlockSpec((G, D), lambda i: (0, 0)),
    out_shape=jax.ShapeDtypeStruct((G, D), _f32),
    scratch_shapes=[pltpu.VMEM((G, D), _f32), pltpu.VMEM((G, D), _f32)],
)


def kernel(node_ids, edge_index, batch, emb,
           W1, a_src1, a_dst1, b1,
           W2, a_src2, a_dst2, b2,
           W3, a_src3, a_dst3, b3):
    ids_p = jnp.pad(node_ids.astype(_i32), (0, NP - N))
    src = jnp.pad(edge_index[0].astype(_i32), (0, EP - E), constant_values=NP - 2)
    dst = jnp.pad(edge_index[1].astype(_i32), (0, EP - E), constant_values=NP - 1)
    batch_p = jnp.pad(batch.astype(_i32), (0, NP - N), constant_values=G)
    batch_p = batch_p.reshape(NP, 1)

    x = _emb_gather(ids_p, emb)

    h1, s1, d1 = _mm_alpha(x, W1, a_src1.reshape(1, D), a_dst1.reshape(1, D))
    f1, e1 = _edge_accum(h1, s1.reshape(NP), d1.reshape(NP), src, dst)
    e1 = e1.reshape(2, NP, 1)

    h2, s2, d2 = _comb_mm(f1, e1, h1, s1, d1, b1.reshape(1, D),
                          W2, a_src2.reshape(1, D), a_dst2.reshape(1, D))
    f2, e2 = _edge_accum(h2, s2.reshape(NP), d2.reshape(NP), src, dst)
    e2 = e2.reshape(2, NP, 1)

    h3, s3, d3 = _comb_mm(f2, e2, h2, s2, d2, b2.reshape(1, D),
                          W3, a_src3.reshape(1, D), a_dst3.reshape(1, D))
    f3, e3 = _edge_accum(h3, s3.reshape(NP), d3.reshape(NP), src, dst)
    e3 = e3.reshape(2, NP, 1)

    return _pool(f3, e3, h3, s3, d3, b3.reshape(1, D), batch_p)


# speedup vs baseline: 19.0671x; 1.6388x over previous
import functools

import jax
import jax.numpy as jnp
from jax import lax
from jax.experimental import pallas as pl
from jax.experimental.pallas import tpu as pltpu
from jax.experimental.pallas import tpu_sc as plsc

N = 10000
NP = 10240
E = 320000
D = 128
G = 64
NW = 32
K = 64
NCH = 158
EP = NW * K * NCH
EPW = EP // NW
RPW = NP // NW
ZR = NP // 16
DR = NP // D

_mesh = plsc.VectorSubcoreMesh(core_axis_name="c", subcore_axis_name="s")
_f32 = jnp.float32
_i32 = jnp.int32
_sc_params = pltpu.CompilerParams(needs_layout_passes=False)


@functools.partial(
    pl.kernel,
    out_type=jax.ShapeDtypeStruct((NP, D), _f32),
    mesh=_mesh,
    scratch_types=[
        pltpu.VMEM((K,), _i32),
        pltpu.VMEM((K, D), _f32),
        pltpu.SemaphoreType.DMA,
    ],
    compiler_params=_sc_params,
)
def _emb_gather(ids_hbm, emb_hbm, x_hbm, idx_v, rows_v, sem):
    c = lax.axis_index("c")
    s = lax.axis_index("s")
    wid = s * 2 + c
    base = wid * RPW
    for ch in range(RPW // K):
        off = base + ch * K
        pltpu.sync_copy(ids_hbm.at[pl.ds(off, K)], idx_v)
        pltpu.async_copy(emb_hbm.at[idx_v], rows_v, sem).wait()
        pltpu.sync_copy(rows_v, x_hbm.at[pl.ds(off, K)])


@functools.partial(
    pl.kernel,
    out_type=(
        jax.ShapeDtypeStruct((2, NP, D), _f32),
        jax.ShapeDtypeStruct((2, DR, D), _f32),
    ),
    mesh=_mesh,
    scratch_types=[
        pltpu.VMEM((NP,), _f32),
        pltpu.VMEM((NP,), _f32),
        pltpu.VMEM((K,), _i32),
        pltpu.VMEM((K,), _i32),
        pltpu.VMEM((K,), _i32),
        pltpu.VMEM((K,), _i32),
        pltpu.VMEM((K,), _i32),
        pltpu.VMEM((K,), _i32),
        pltpu.VMEM((K,), _f32),
        pltpu.VMEM((DR,), _i32),
        pltpu.VMEM((DR, D), _f32),
        pltpu.VMEM((K, D), _f32),
        pltpu.VMEM((K, D), _f32),
        pltpu.VMEM_SHARED((NP, D), _f32),
        pltpu.VMEM_SHARED((DR, D), _f32),
        pltpu.SemaphoreType.DMA,
        pltpu.SemaphoreType.DMA,
        pltpu.SemaphoreType.DMA,
        pltpu.SemaphoreType.DMA,
        pltpu.SemaphoreType.DMA,
        pltpu.SemaphoreType.DMA,
    ],
    compiler_params=_sc_params,
)
def _edge_accum(h_hbm, as_hbm, ad_hbm, src_hbm, dst_hbm, feat_hbm, den_hbm,
                asv, adv, srcv0, srcv1, dstv0, dstv1, sdst0, sdst1, exv, idxr,
                denv, rows0, rows1, feat_sh, den_sh,
                ese0, ese1, gse0, gse1, sse0, sse1):
    c = lax.axis_index("c")
    s = lax.axis_index("s")
    wid = s * 2 + c
    zero16 = jnp.zeros((16,), _f32)
    iota16 = lax.iota(_i32, 16)
    srcv = (srcv0, srcv1)
    dstv = (dstv0, dstv1)
    sdst = (sdst0, sdst1)
    rows = (rows0, rows1)
    ese = (ese0, ese1)
    gse = (gse0, gse1)
    sse = (sse0, sse1)
    ebase = wid * EPW

    def _zb(i, _):
        r = i // (D // 16)
        col = (i % (D // 16)) * 16
        rows0[r, pl.ds(col, 16)] = zero16
        return 0
    lax.fori_loop(0, K * D // 16, _zb, 0)

    def _zd(i, _):
        r = i // (D // 16)
        col = (i % (D // 16)) * 16
        denv[r, pl.ds(col, 16)] = zero16
        return 0
    lax.fori_loop(0, DR * D // 16, _zd, 0)
    for g in range(DR // 16):
        idxr[pl.ds(g * 16, 16)] = iota16 + g * 16

    zbase = s * ZR

    def _zsh(i, _):
        pltpu.sync_copy(rows0, feat_sh.at[pl.ds(zbase + i * K, K)])
        return 0
    lax.fori_loop(0, ZR // K, _zsh, 0)

    @pl.when(s == 0)
    def _():
        pltpu.sync_copy(denv, den_sh)

    pltpu.sync_copy(as_hbm, asv)
    pltpu.sync_copy(ad_hbm, adv)
    plsc.subcore_barrier()

    def _edges_start(ci, b):
        off = ebase + ci * K
        pltpu.async_copy(src_hbm.at[pl.ds(off, K)], srcv[b], ese[b])
        pltpu.async_copy(dst_hbm.at[pl.ds(off, K)], dstv[b], ese[b])

    def _edges_wait(ci, b):
        off = ebase + ci * K
        pltpu.make_async_copy(src_hbm.at[pl.ds(off, K)], srcv[b], ese[b]).wait()
        pltpu.make_async_copy(dst_hbm.at[pl.ds(off, K)], dstv[b], ese[b]).wait()

    _edges_start(0, 0)
    _edges_start(1, 1)
    _edges_wait(0, 0)
    pltpu.async_copy(h_hbm.at[srcv[0]], rows[0], gse[0])

    def _iter(ci, b):
        bn = 1 - b
        pltpu.make_async_copy(h_hbm.at[srcv[b]], rows[b], gse[b]).wait()
        for g in range(K // 16):
            si = srcv[b][pl.ds(g * 16, 16)]
            di = dstv[b][pl.ds(g * 16, 16)]
            e = plsc.load_gather(asv, [si]) + plsc.load_gather(adv, [di])
            e = jnp.where(e >= 0, e, 0.2 * e)
            ex = jnp.exp(e)
            exv[pl.ds(g * 16, 16)] = ex
            sdst[b][pl.ds(g * 16, 16)] = di
            plsc.addupdate_scatter(
                denv,
                [lax.shift_right_logical(di, 7), lax.bitwise_and(di, 127)],
                ex)

        def _scale(i, _):
            exi = plsc.load_gather(exv, [jnp.zeros((16,), _i32) + i])
            for j in range(D // 16):
                rows[b][i, pl.ds(j * 16, 16)] *= exi
            return 0
        lax.fori_loop(0, K, _scale, 0)
        pltpu.async_copy(rows[b], feat_sh.at[sdst[b]], sse[b], add=True)
        _edges_start(jnp.minimum(ci + 2, NCH - 1), b)
        _edges_wait(jnp.minimum(ci + 1, NCH - 1), bn)

        @pl.when(ci >= 1)
        def _():
            pltpu.make_async_copy(rows[bn], feat_sh.at[sdst[bn]], sse[bn]).wait()
        pltpu.async_copy(h_hbm.at[srcv[bn]], rows[bn], gse[bn])
        return ()

    def _pair(t, _):
        _iter(2 * t, 0)
        _iter(2 * t + 1, 1)
        return 0
    lax.fori_loop(0, NCH // 2, _pair, 0)

    bl = (NCH - 1) % 2
    pltpu.make_async_copy(h_hbm.at[srcv[1 - bl]], rows[1 - bl], gse[1 - bl]).wait()
    _edges_wait(NCH - 1, bl)
    pltpu.make_async_copy(rows[bl], feat_sh.at[sdst[bl]], sse[bl]).wait()

    plsc.subcore_barrier()
    pltpu.sync_copy(denv, den_sh.at[idxr], add=True)
    plsc.subcore_barrier()

    def _wout(i, _):
        pltpu.sync_copy(feat_sh.at[pl.ds(zbase + i * K, K)], rows0)
        pltpu.sync_copy(rows0, feat_hbm.at[c, pl.ds(zbase + i * K, K)])
        return 0
    lax.fori_loop(0, ZR // K, _wout, 0)

    @pl.when(s == 0)
    def _():
        pltpu.sync_copy(den_sh, rows1.at[pl.ds(0, DR)])
        pltpu.sync_copy(rows1.at[pl.ds(0, DR)], den_hbm.at[c])


_BLK = 1024
_NBLK = NP // _BLK


def _alphas(h, asr, adr):
    a_s = (h * asr).sum(-1, keepdims=True)
    a_d = (h * adr).sum(-1, keepdims=True)
    return a_s, a_d


def _mm_alpha_body(x_ref, w_ref, asr_ref, adr_ref, h_ref, als_ref, ald_ref):
    h = jnp.dot(x_ref[...], w_ref[...], preferred_element_type=_f32)
    h_ref[...] = h
    a_s, a_d = _alphas(h, asr_ref[...], adr_ref[...])
    als_ref[...] = a_s
    ald_ref[...] = a_d


def _combine(feat, den, h, a_s, a_d, b):
    exs = jnp.exp(jnp.where(a_s + a_d >= 0, a_s + a_d, 0.2 * (a_s + a_d)))
    num = feat[0] + feat[1] + exs * h
    dn = den[0] + den[1] + exs + 1e-16
    return num / dn + b


def _comb_mm_body(feat_ref, den_ref, h_ref, als_ref, ald_ref, b_ref,
                  w_ref, asr_ref, adr_ref, hn_ref, alsn_ref, aldn_ref):
    y = _combine(feat_ref[...], den_ref[...], h_ref[...], als_ref[...],
                 ald_ref[...], b_ref[...])
    y = jnp.maximum(y, 0.0)
    hn = jnp.dot(y, w_ref[...], preferred_element_type=_f32)
    hn_ref[...] = hn
    a_s, a_d = _alphas(hn, asr_ref[...], adr_ref[...])
    alsn_ref[...] = a_s
    aldn_ref[...] = a_d


def _pool_body(feat_ref, den_ref, h_ref, als_ref, ald_ref, b_ref, batch_ref,
               out_ref, acc, cnt):
    i = pl.program_id(0)

    @pl.when(i == 0)
    def _():
        acc[...] = jnp.zeros((G, D), _f32)
        cnt[...] = jnp.zeros((G, D), _f32)

    y = _combine(feat_ref[...], den_ref[...], h_ref[...], als_ref[...],
                 ald_ref[...], b_ref[...])
    oh = (batch_ref[...] == lax.broadcasted_iota(_i32, (1, G), 1)).astype(_f32)
    dn = (((0,), (0,)), ((), ()))
    acc[...] += lax.dot_general(oh, y, dn, preferred_element_type=_f32)
    cnt[...] += lax.dot_general(oh, jnp.ones_like(y), dn, preferred_element_type=_f32)

    @pl.when(i == _NBLK - 1)
    def _():
        out_ref[...] = acc[...] / jnp.maximum(cnt[...], 1.0)


_row_spec = pl.BlockSpec((_BLK, D), lambda i: (i, 0))
_col_spec = pl.BlockSpec((_BLK, 1), lambda i: (i, 0))
_w_spec = pl.BlockSpec((D, D), lambda i: (0, 0))
_v_spec = pl.BlockSpec((1, D), lambda i: (0, 0))
_feat_spec = pl.BlockSpec((2, _BLK, D), lambda i: (0, i, 0))
_den_spec = pl.BlockSpec((2, _BLK, 1), lambda i: (0, i, 0))

_mm_alpha = pl.pallas_call(
    _mm_alpha_body,
    grid=(_NBLK,),
    in_specs=[_row_spec, _w_spec, _v_spec, _v_spec],
    out_specs=[_row_spec, _col_spec, _col_spec],
    out_shape=[
        jax.ShapeDtypeStruct((NP, D), _f32),
        jax.ShapeDtypeStruct((NP, 1), _f32),
        jax.ShapeDtypeStruct((NP, 1), _f32),
    ],
)

_comb_mm = pl.pallas_call(
    _comb_mm_body,
    grid=(_NBLK,),
    in_specs=[_feat_spec, _den_spec, _row_spec, _col_spec, _col_spec, _v_spec,
              _w_spec, _v_spec, _v_spec],
    out_specs=[_row_spec, _col_spec, _col_spec],
    out_shape=[
        jax.ShapeDtypeStruct((NP, D), _f32),
        jax.ShapeDtypeStruct((NP, 1), _f32),
        jax.ShapeDtypeStruct((NP, 1), _f32),
    ],
)

_pool = pl.pallas_call(
    _pool_body,
    grid=(_NBLK,),
    in_specs=[_feat_spec, _den_spec, _row_spec, _col_spec, _col_spec, _v_spec,
              pl.BlockSpec((_BLK, 1), lambda i: (i, 0))],
    out_specs=pl.BlockSpec((G, D), lambda i: (0, 0)),
    out_shape=jax.ShapeDtypeStruct((G, D), _f32),
    scratch_shapes=[pltpu.VMEM((G, D), _f32), pltpu.VMEM((G, D), _f32)],
)


def kernel(node_ids, edge_index, batch, emb,
           W1, a_src1, a_dst1, b1,
           W2, a_src2, a_dst2, b2,
           W3, a_src3, a_dst3, b3):
    ids_p = jnp.pad(node_ids.astype(_i32), (0, NP - N))
    src = jnp.pad(edge_index[0].astype(_i32), (0, EP - E), constant_values=NP - 2)
    dst = jnp.pad(edge_index[1].astype(_i32), (0, EP - E), constant_values=NP - 1)
    batch_p = jnp.pad(batch.astype(_i32), (0, NP - N), constant_values=G)
    batch_p = batch_p.reshape(NP, 1)

    x = _emb_gather(ids_p, emb)

    h1, s1, d1 = _mm_alpha(x, W1, a_src1.reshape(1, D), a_dst1.reshape(1, D))
    f1, e1 = _edge_accum(h1, s1.reshape(NP), d1.reshape(NP), src, dst)
    e1 = e1.reshape(2, NP, 1)

    h2, s2, d2 = _comb_mm(f1, e1, h1, s1, d1, b1.reshape(1, D),
                          W2, a_src2.reshape(1, D), a_dst2.reshape(1, D))
    f2, e2 = _edge_accum(h2, s2.reshape(NP), d2.reshape(NP), src, dst)
    e2 = e2.reshape(2, NP, 1)

    h3, s3, d3 = _comb_mm(f2, e2, h2, s2, d2, b2.reshape(1, D),
                          W3, a_src3.reshape(1, D), a_dst3.reshape(1, D))
    f3, e3 = _edge_accum(h3, s3.reshape(NP), d3.reshape(NP), src, dst)
    e3 = e3.reshape(2, NP, 1)

    return _pool(f3, e3, h3, s3, d3, b3.reshape(1, D), batch_p)


# speedup vs baseline: 21.0405x; 1.1035x over previous
import functools

import jax
import jax.numpy as jnp
from jax import lax
from jax.experimental import pallas as pl
from jax.experimental.pallas import tpu as pltpu
from jax.experimental.pallas import tpu_sc as plsc

N = 10000
NP = 10240
E = 320000
D = 128
G = 64
NW = 32
K = 64
NCH = 158
EP = NW * K * NCH
EPW = EP // NW
RPW = NP // NW
ZR = NP // 16
DR = NP // D

_mesh = plsc.VectorSubcoreMesh(core_axis_name="c", subcore_axis_name="s")
_f32 = jnp.float32
_i32 = jnp.int32
_sc_params = pltpu.CompilerParams(needs_layout_passes=False)


@functools.partial(
    pl.kernel,
    out_type=jax.ShapeDtypeStruct((NP, D), _f32),
    mesh=_mesh,
    scratch_types=[
        pltpu.VMEM((K,), _i32),
        pltpu.VMEM((K, D), _f32),
        pltpu.SemaphoreType.DMA,
    ],
    compiler_params=_sc_params,
)
def _emb_gather(ids_hbm, emb_hbm, x_hbm, idx_v, rows_v, sem):
    c = lax.axis_index("c")
    s = lax.axis_index("s")
    wid = s * 2 + c
    base = wid * RPW
    for ch in range(RPW // K):
        off = base + ch * K
        pltpu.sync_copy(ids_hbm.at[pl.ds(off, K)], idx_v)
        pltpu.async_copy(emb_hbm.at[idx_v], rows_v, sem).wait()
        pltpu.sync_copy(rows_v, x_hbm.at[pl.ds(off, K)])


@functools.partial(
    pl.kernel,
    out_type=(
        jax.ShapeDtypeStruct((2, NP, D), _f32),
        jax.ShapeDtypeStruct((2, DR, D), _f32),
    ),
    mesh=_mesh,
    scratch_types=[
        pltpu.VMEM((NP,), _f32),
        pltpu.VMEM((NP,), _f32),
        pltpu.VMEM((K,), _i32),
        pltpu.VMEM((K,), _i32),
        pltpu.VMEM((K,), _i32),
        pltpu.VMEM((K,), _i32),
        pltpu.VMEM((K,), _i32),
        pltpu.VMEM((K,), _i32),
        pltpu.VMEM((K,), _f32),
        pltpu.VMEM((DR,), _i32),
        pltpu.VMEM((DR, D), _f32),
        pltpu.VMEM((K, D), _f32),
        pltpu.VMEM((K, D), _f32),
        pltpu.VMEM_SHARED((NP, D), _f32),
        pltpu.VMEM_SHARED((DR, D), _f32),
        pltpu.SemaphoreType.DMA,
        pltpu.SemaphoreType.DMA,
        pltpu.SemaphoreType.DMA,
        pltpu.SemaphoreType.DMA,
        pltpu.SemaphoreType.DMA,
        pltpu.SemaphoreType.DMA,
    ],
    compiler_params=_sc_params,
)
def _edge_accum(h_hbm, as_hbm, ad_hbm, src_hbm, dst_hbm, feat_hbm, den_hbm,
                asv, adv, srcv0, srcv1, dstv0, dstv1, sdst0, sdst1, exv, idxr,
                denv, rows0, rows1, feat_sh, den_sh,
                ese0, ese1, gse0, gse1, sse0, sse1):
    c = lax.axis_index("c")
    s = lax.axis_index("s")
    wid = s * 2 + c
    zero16 = jnp.zeros((16,), _f32)
    iota16 = lax.iota(_i32, 16)
    srcv = (srcv0, srcv1)
    dstv = (dstv0, dstv1)
    sdst = (sdst0, sdst1)
    rows = (rows0, rows1)
    ese = (ese0, ese1)
    gse = (gse0, gse1)
    sse = (sse0, sse1)
    ebase = wid * EPW

    def _zb(i, _):
        r = i // (D // 16)
        col = (i % (D // 16)) * 16
        rows0[r, pl.ds(col, 16)] = zero16
        return 0
    lax.fori_loop(0, K * D // 16, _zb, 0)

    def _zd(i, _):
        r = i // (D // 16)
        col = (i % (D // 16)) * 16
        denv[r, pl.ds(col, 16)] = zero16
        return 0
    lax.fori_loop(0, DR * D // 16, _zd, 0)
    for g in range(DR // 16):
        idxr[pl.ds(g * 16, 16)] = iota16 + g * 16

    zbase = s * ZR

    def _zsh(i, _):
        pltpu.sync_copy(rows0, feat_sh.at[pl.ds(zbase + i * K, K)])
        return 0
    lax.fori_loop(0, ZR // K, _zsh, 0)

    @pl.when(s == 0)
    def _():
        pltpu.sync_copy(denv, den_sh)

    pltpu.sync_copy(as_hbm, asv)
    pltpu.sync_copy(ad_hbm, adv)
    plsc.subcore_barrier()

    def _edges_start(ci, b):
        off = ebase + ci * K
        pltpu.async_copy(src_hbm.at[pl.ds(off, K)], srcv[b], ese[b])
        pltpu.async_copy(dst_hbm.at[pl.ds(off, K)], dstv[b], ese[b])

    def _edges_wait(ci, b):
        off = ebase + ci * K
        pltpu.make_async_copy(src_hbm.at[pl.ds(off, K)], srcv[b], ese[b]).wait()
        pltpu.make_async_copy(dst_hbm.at[pl.ds(off, K)], dstv[b], ese[b]).wait()

    _edges_start(0, 0)
    _edges_start(1, 1)
    _edges_wait(0, 0)
    pltpu.async_copy(h_hbm.at[srcv[0]], rows[0], gse[0])

    def _iter(ci, b):
        bn = 1 - b
        pltpu.make_async_copy(h_hbm.at[srcv[b]], rows[b], gse[b]).wait()
        for g in range(K // 16):
            si = srcv[b][pl.ds(g * 16, 16)]
            di = dstv[b][pl.ds(g * 16, 16)]
            e = plsc.load_gather(asv, [si]) + plsc.load_gather(adv, [di])
            e = jnp.where(e >= 0, e, 0.2 * e)
            ex = jnp.exp(e)
            exv[pl.ds(g * 16, 16)] = ex
            sdst[b][pl.ds(g * 16, 16)] = di
            plsc.addupdate_scatter(
                denv,
                [lax.shift_right_logical(di, 7), lax.bitwise_and(di, 127)],
                ex)

        @plsc.parallel_loop(0, K, unroll=4)
        def _(i):
            exi = plsc.load_gather(exv, [jnp.zeros((16,), _i32) + i])
            for j in range(D // 16):
                rows[b][i, pl.ds(j * 16, 16)] *= exi
        pltpu.async_copy(rows[b], feat_sh.at[sdst[b]], sse[b], add=True)
        _edges_start(jnp.minimum(ci + 2, NCH - 1), b)
        _edges_wait(jnp.minimum(ci + 1, NCH - 1), bn)

        @pl.when(ci >= 1)
        def _():
            pltpu.make_async_copy(rows[bn], feat_sh.at[sdst[bn]], sse[bn]).wait()
        pltpu.async_copy(h_hbm.at[srcv[bn]], rows[bn], gse[bn])
        return ()

    def _pair(t, _):
        _iter(2 * t, 0)
        _iter(2 * t + 1, 1)
        return 0
    lax.fori_loop(0, NCH // 2, _pair, 0)

    bl = (NCH - 1) % 2
    pltpu.make_async_copy(h_hbm.at[srcv[1 - bl]], rows[1 - bl], gse[1 - bl]).wait()
    _edges_wait(NCH - 1, bl)
    pltpu.make_async_copy(rows[bl], feat_sh.at[sdst[bl]], sse[bl]).wait()

    plsc.subcore_barrier()
    pltpu.sync_copy(denv, den_sh.at[idxr], add=True)
    plsc.subcore_barrier()

    def _wout(i, _):
        pltpu.sync_copy(feat_sh.at[pl.ds(zbase + i * K, K)], rows0)
        pltpu.sync_copy(rows0, feat_hbm.at[c, pl.ds(zbase + i * K, K)])
        return 0
    lax.fori_loop(0, ZR // K, _wout, 0)

    @pl.when(s == 0)
    def _():
        pltpu.sync_copy(den_sh, rows1.at[pl.ds(0, DR)])
        pltpu.sync_copy(rows1.at[pl.ds(0, DR)], den_hbm.at[c])


_BLK = 1024
_NBLK = NP // _BLK


def _alphas(h, asr, adr):
    a_s = (h * asr).sum(-1, keepdims=True)
    a_d = (h * adr).sum(-1, keepdims=True)
    return a_s, a_d


def _mm_alpha_body(x_ref, w_ref, asr_ref, adr_ref, h_ref, als_ref, ald_ref):
    h = jnp.dot(x_ref[...], w_ref[...], preferred_element_type=_f32)
    h_ref[...] = h
    a_s, a_d = _alphas(h, asr_ref[...], adr_ref[...])
    als_ref[...] = a_s
    ald_ref[...] = a_d


def _combine(feat, den, h, a_s, a_d, b):
    exs = jnp.exp(jnp.where(a_s + a_d >= 0, a_s + a_d, 0.2 * (a_s + a_d)))
    num = feat[0] + feat[1] + exs * h
    dn = den[0] + den[1] + exs + 1e-16
    return num / dn + b


def _comb_mm_body(feat_ref, den_ref, h_ref, als_ref, ald_ref, b_ref,
                  w_ref, asr_ref, adr_ref, hn_ref, alsn_ref, aldn_ref):
    y = _combine(feat_ref[...], den_ref[...], h_ref[...], als_ref[...],
                 ald_ref[...], b_ref[...])
    y = jnp.maximum(y, 0.0)
    hn = jnp.dot(y, w_ref[...], preferred_element_type=_f32)
    hn_ref[...] = hn
    a_s, a_d = _alphas(hn, asr_ref[...], adr_ref[...])
    alsn_ref[...] = a_s
    aldn_ref[...] = a_d


def _pool_body(feat_ref, den_ref, h_ref, als_ref, ald_ref, b_ref, batch_ref,
               out_ref, acc, cnt):
    i = pl.program_id(0)

    @pl.when(i == 0)
    def _():
        acc[...] = jnp.zeros((G, D), _f32)
        cnt[...] = jnp.zeros((G, D), _f32)

    y = _combine(feat_ref[...], den_ref[...], h_ref[...], als_ref[...],
                 ald_ref[...], b_ref[...])
    oh = (batch_ref[...] == lax.broadcasted_iota(_i32, (1, G), 1)).astype(_f32)
    dn = (((0,), (0,)), ((), ()))
    acc[...] += lax.dot_general(oh, y, dn, preferred_element_type=_f32)
    cnt[...] += lax.dot_general(oh, jnp.ones_like(y), dn, preferred_element_type=_f32)

    @pl.when(i == _NBLK - 1)
    def _():
        out_ref[...] = acc[...] / jnp.maximum(cnt[...], 1.0)


_row_spec = pl.BlockSpec((_BLK, D), lambda i: (i, 0))
_col_spec = pl.BlockSpec((_BLK, 1), lambda i: (i, 0))
_w_spec = pl.BlockSpec((D, D), lambda i: (0, 0))
_v_spec = pl.BlockSpec((1, D), lambda i: (0, 0))
_feat_spec = pl.BlockSpec((2, _BLK, D), lambda i: (0, i, 0))
_den_spec = pl.BlockSpec((2, _BLK, 1), lambda i: (0, i, 0))

_mm_alpha = pl.pallas_call(
    _mm_alpha_body,
    grid=(_NBLK,),
    in_specs=[_row_spec, _w_spec, _v_spec, _v_spec],
    out_specs=[_row_spec, _col_spec, _col_spec],
    out_shape=[
        jax.ShapeDtypeStruct((NP, D), _f32),
        jax.ShapeDtypeStruct((NP, 1), _f32),
        jax.ShapeDtypeStruct((NP, 1), _f32),
    ],
)

_comb_mm = pl.pallas_call(
    _comb_mm_body,
    grid=(_NBLK,),
    in_specs=[_feat_spec, _den_spec, _row_spec, _col_spec, _col_spec, _v_spec,
              _w_spec, _v_spec, _v_spec],
    out_specs=[_row_spec, _col_spec, _col_spec],
    out_shape=[
        jax.ShapeDtypeStruct((NP, D), _f32),
        jax.ShapeDtypeStruct((NP, 1), _f32),
        jax.ShapeDtypeStruct((NP, 1), _f32),
    ],
)

_pool = pl.pallas_call(
    _pool_body,
    grid=(_NBLK,),
    in_specs=[_feat_spec, _den_spec, _row_spec, _col_spec, _col_spec, _v_spec,
              pl.BlockSpec((_BLK, 1), lambda i: (i, 0))],
    out_specs=pl.BlockSpec((G, D), lambda i: (0, 0)),
    out_shape=jax.ShapeDtypeStruct((G, D), _f32),
    scratch_shapes=[pltpu.VMEM((G, D), _f32), pltpu.VMEM((G, D), _f32)],
)


def kernel(node_ids, edge_index, batch, emb,
           W1, a_src1, a_dst1, b1,
           W2, a_src2, a_dst2, b2,
           W3, a_src3, a_dst3, b3):
    ids_p = jnp.pad(node_ids.astype(_i32), (0, NP - N))
    src = jnp.pad(edge_index[0].astype(_i32), (0, EP - E), constant_values=NP - 2)
    dst = jnp.pad(edge_index[1].astype(_i32), (0, EP - E), constant_values=NP - 1)
    batch_p = jnp.pad(batch.astype(_i32), (0, NP - N), constant_values=G)
    batch_p = batch_p.reshape(NP, 1)

    x = _emb_gather(ids_p, emb)

    h1, s1, d1 = _mm_alpha(x, W1, a_src1.reshape(1, D), a_dst1.reshape(1, D))
    f1, e1 = _edge_accum(h1, s1.reshape(NP), d1.reshape(NP), src, dst)
    e1 = e1.reshape(2, NP, 1)

    h2, s2, d2 = _comb_mm(f1, e1, h1, s1, d1, b1.reshape(1, D),
                          W2, a_src2.reshape(1, D), a_dst2.reshape(1, D))
    f2, e2 = _edge_accum(h2, s2.reshape(NP), d2.reshape(NP), src, dst)
    e2 = e2.reshape(2, NP, 1)

    h3, s3, d3 = _comb_mm(f2, e2, h2, s2, d2, b2.reshape(1, D),
                          W3, a_src3.reshape(1, D), a_dst3.reshape(1, D))
    f3, e3 = _edge_accum(h3, s3.reshape(NP), d3.reshape(NP), src, dst)
    e3 = e3.reshape(2, NP, 1)

    return _pool(f3, e3, h3, s3, d3, b3.reshape(1, D), batch_p)


# speedup vs baseline: 21.0580x; 1.0008x over previous
import functools

import jax
import jax.numpy as jnp
from jax import lax
from jax.experimental import pallas as pl
from jax.experimental.pallas import tpu as pltpu
from jax.experimental.pallas import tpu_sc as plsc

N = 10000
NP = 10240
E = 320000
D = 128
G = 64
NW = 32
K = 64
NCH = 158
EP = NW * K * NCH
EPW = EP // NW
RPW = NP // NW
ZR = NP // 16
DR = NP // D

_mesh = plsc.VectorSubcoreMesh(core_axis_name="c", subcore_axis_name="s")
_f32 = jnp.float32
_i32 = jnp.int32
_sc_params = pltpu.CompilerParams(needs_layout_passes=False)


@functools.partial(
    pl.kernel,
    out_type=jax.ShapeDtypeStruct((NP, D), _f32),
    mesh=_mesh,
    scratch_types=[
        pltpu.VMEM((K,), _i32),
        pltpu.VMEM((K, D), _f32),
        pltpu.SemaphoreType.DMA,
    ],
    compiler_params=_sc_params,
)
def _emb_gather(ids_hbm, emb_hbm, x_hbm, idx_v, rows_v, sem):
    c = lax.axis_index("c")
    s = lax.axis_index("s")
    wid = s * 2 + c
    base = wid * RPW
    for ch in range(RPW // K):
        off = base + ch * K
        pltpu.sync_copy(ids_hbm.at[pl.ds(off, K)], idx_v)
        pltpu.async_copy(emb_hbm.at[idx_v], rows_v, sem).wait()
        pltpu.sync_copy(rows_v, x_hbm.at[pl.ds(off, K)])


@functools.partial(
    pl.kernel,
    out_type=(
        jax.ShapeDtypeStruct((2, NP, D), _f32),
        jax.ShapeDtypeStruct((2, DR, D), _f32),
    ),
    mesh=_mesh,
    scratch_types=[
        pltpu.VMEM((NP,), _f32),
        pltpu.VMEM((NP,), _f32),
        pltpu.VMEM((K,), _i32),
        pltpu.VMEM((K,), _i32),
        pltpu.VMEM((K,), _i32),
        pltpu.VMEM((K,), _i32),
        pltpu.VMEM((K,), _i32),
        pltpu.VMEM((K,), _i32),
        pltpu.VMEM((K,), _f32),
        pltpu.VMEM((DR,), _i32),
        pltpu.VMEM((DR, D), _f32),
        pltpu.VMEM((K, D), _f32),
        pltpu.VMEM((K, D), _f32),
        pltpu.VMEM_SHARED((NP, D), _f32),
        pltpu.VMEM_SHARED((DR, D), _f32),
        pltpu.SemaphoreType.DMA,
        pltpu.SemaphoreType.DMA,
        pltpu.SemaphoreType.DMA,
        pltpu.SemaphoreType.DMA,
        pltpu.SemaphoreType.DMA,
        pltpu.SemaphoreType.DMA,
    ],
    compiler_params=_sc_params,
)
def _edge_accum(h_hbm, as_hbm, ad_hbm, src_hbm, dst_hbm, feat_hbm, den_hbm,
                asv, adv, srcv0, srcv1, dstv0, dstv1, sdst0, sdst1, exv, idxr,
                denv, rows0, rows1, feat_sh, den_sh,
                ese0, ese1, gse0, gse1, sse0, sse1):
    c = lax.axis_index("c")
    s = lax.axis_index("s")
    wid = s * 2 + c
    zero16 = jnp.zeros((16,), _f32)
    iota16 = lax.iota(_i32, 16)
    srcv = (srcv0, srcv1)
    dstv = (dstv0, dstv1)
    sdst = (sdst0, sdst1)
    rows = (rows0, rows1)
    ese = (ese0, ese1)
    gse = (gse0, gse1)
    sse = (sse0, sse1)
    ebase = wid * EPW

    def _zb(i, _):
        r = i // (D // 16)
        col = (i % (D // 16)) * 16
        rows0[r, pl.ds(col, 16)] = zero16
        return 0
    lax.fori_loop(0, K * D // 16, _zb, 0)

    def _zd(i, _):
        r = i // (D // 16)
        col = (i % (D // 16)) * 16
        denv[r, pl.ds(col, 16)] = zero16
        return 0
    lax.fori_loop(0, DR * D // 16, _zd, 0)
    for g in range(DR // 16):
        idxr[pl.ds(g * 16, 16)] = iota16 + g * 16

    zbase = s * ZR

    def _zsh(i, _):
        pltpu.sync_copy(rows0, feat_sh.at[pl.ds(zbase + i * K, K)])
        return 0
    lax.fori_loop(0, ZR // K, _zsh, 0)

    @pl.when(s == 0)
    def _():
        pltpu.sync_copy(denv, den_sh)

    pltpu.sync_copy(as_hbm, asv)
    pltpu.sync_copy(ad_hbm, adv)
    plsc.subcore_barrier()

    def _edges_start(ci, b):
        off = ebase + ci * K
        pltpu.async_copy(src_hbm.at[pl.ds(off, K)], srcv[b], ese[b])
        pltpu.async_copy(dst_hbm.at[pl.ds(off, K)], dstv[b], ese[b])

    def _edges_wait(ci, b):
        off = ebase + ci * K
        pltpu.make_async_copy(src_hbm.at[pl.ds(off, K)], srcv[b], ese[b]).wait()
        pltpu.make_async_copy(dst_hbm.at[pl.ds(off, K)], dstv[b], ese[b]).wait()

    def _gather_start(b):
        for hh in range(2):
            pltpu.async_copy(
                h_hbm.at[srcv[b].at[pl.ds(hh * (K // 2), K // 2)]],
                rows[b].at[pl.ds(hh * (K // 2), K // 2)], gse[b])

    def _gather_wait(b):
        for hh in range(2):
            pltpu.make_async_copy(
                h_hbm.at[srcv[b].at[pl.ds(hh * (K // 2), K // 2)]],
                rows[b].at[pl.ds(hh * (K // 2), K // 2)], gse[b]).wait()

    _edges_start(0, 0)
    _edges_start(1, 1)
    _edges_wait(0, 0)
    _gather_start(0)

    def _iter(ci, b):
        bn = 1 - b
        _gather_wait(b)
        for g in range(K // 16):
            si = srcv[b][pl.ds(g * 16, 16)]
            di = dstv[b][pl.ds(g * 16, 16)]
            e = plsc.load_gather(asv, [si]) + plsc.load_gather(adv, [di])
            e = jnp.where(e >= 0, e, 0.2 * e)
            ex = jnp.exp(e)
            exv[pl.ds(g * 16, 16)] = ex
            sdst[b][pl.ds(g * 16, 16)] = di
            plsc.addupdate_scatter(
                denv,
                [lax.shift_right_logical(di, 7), lax.bitwise_and(di, 127)],
                ex)

        @plsc.parallel_loop(0, K, unroll=4)
        def _(i):
            exi = plsc.load_gather(exv, [jnp.zeros((16,), _i32) + i])
            for j in range(D // 16):
                rows[b][i, pl.ds(j * 16, 16)] *= exi
        pltpu.async_copy(rows[b], feat_sh.at[sdst[b]], sse[b], add=True)
        _edges_start(jnp.minimum(ci + 2, NCH - 1), b)
        _edges_wait(jnp.minimum(ci + 1, NCH - 1), bn)

        @pl.when(ci >= 1)
        def _():
            pltpu.make_async_copy(rows[bn], feat_sh.at[sdst[bn]], sse[bn]).wait()
        _gather_start(bn)
        return ()

    def _pair(t, _):
        _iter(2 * t, 0)
        _iter(2 * t + 1, 1)
        return 0
    lax.fori_loop(0, NCH // 2, _pair, 0)

    bl = (NCH - 1) % 2
    _gather_wait(1 - bl)
    _edges_wait(NCH - 1, bl)
    pltpu.make_async_copy(rows[bl], feat_sh.at[sdst[bl]], sse[bl]).wait()

    plsc.subcore_barrier()
    pltpu.sync_copy(denv, den_sh.at[idxr], add=True)
    plsc.subcore_barrier()

    def _wout(i, _):
        pltpu.sync_copy(feat_sh.at[pl.ds(zbase + i * K, K)], rows0)
        pltpu.sync_copy(rows0, feat_hbm.at[c, pl.ds(zbase + i * K, K)])
        return 0
    lax.fori_loop(0, ZR // K, _wout, 0)

    @pl.when(s == 0)
    def _():
        pltpu.sync_copy(den_sh, rows1.at[pl.ds(0, DR)])
        pltpu.sync_copy(rows1.at[pl.ds(0, DR)], den_hbm.at[c])


_BLK = 1024
_NBLK = NP // _BLK


def _alphas(h, asr, adr):
    a_s = (h * asr).sum(-1, keepdims=True)
    a_d = (h * adr).sum(-1, keepdims=True)
    return a_s, a_d


def _mm_alpha_body(x_ref, w_ref, asr_ref, adr_ref, h_ref, als_ref, ald_ref):
    h = jnp.dot(x_ref[...], w_ref[...], preferred_element_type=_f32)
    h_ref[...] = h
    a_s, a_d = _alphas(h, asr_ref[...], adr_ref[...])
    als_ref[...] = a_s
    ald_ref[...] = a_d


def _combine(feat, den, h, a_s, a_d, b):
    exs = jnp.exp(jnp.where(a_s + a_d >= 0, a_s + a_d, 0.2 * (a_s + a_d)))
    num = feat[0] + feat[1] + exs * h
    dn = den[0] + den[1] + exs + 1e-16
    return num / dn + b


def _comb_mm_body(feat_ref, den_ref, h_ref, als_ref, ald_ref, b_ref,
                  w_ref, asr_ref, adr_ref, hn_ref, alsn_ref, aldn_ref):
    y = _combine(feat_ref[...], den_ref[...], h_ref[...], als_ref[...],
                 ald_ref[...], b_ref[...])
    y = jnp.maximum(y, 0.0)
    hn = jnp.dot(y, w_ref[...], preferred_element_type=_f32)
    hn_ref[...] = hn
    a_s, a_d = _alphas(hn, asr_ref[...], adr_ref[...])
    alsn_ref[...] = a_s
    aldn_ref[...] = a_d


def _pool_body(feat_ref, den_ref, h_ref, als_ref, ald_ref, b_ref, batch_ref,
               out_ref, acc, cnt):
    i = pl.program_id(0)

    @pl.when(i == 0)
    def _():
        acc[...] = jnp.zeros((G, D), _f32)
        cnt[...] = jnp.zeros((G, D), _f32)

    y = _combine(feat_ref[...], den_ref[...], h_ref[...], als_ref[...],
                 ald_ref[...], b_ref[...])
    oh = (batch_ref[...] == lax.broadcasted_iota(_i32, (1, G), 1)).astype(_f32)
    dn = (((0,), (0,)), ((), ()))
    acc[...] += lax.dot_general(oh, y, dn, preferred_element_type=_f32)
    cnt[...] += lax.dot_general(oh, jnp.ones_like(y), dn, preferred_element_type=_f32)

    @pl.when(i == _NBLK - 1)
    def _():
        out_ref[...] = acc[...] / jnp.maximum(cnt[...], 1.0)


_row_spec = pl.BlockSpec((_BLK, D), lambda i: (i, 0))
_col_spec = pl.BlockSpec((_BLK, 1), lambda i: (i, 0))
_w_spec = pl.BlockSpec((D, D), lambda i: (0, 0))
_v_spec = pl.BlockSpec((1, D), lambda i: (0, 0))
_feat_spec = pl.BlockSpec((2, _BLK, D), lambda i: (0, i, 0))
_den_spec = pl.BlockSpec((2, _BLK, 1), lambda i: (0, i, 0))

_mm_alpha = pl.pallas_call(
    _mm_alpha_body,
    grid=(_NBLK,),
    in_specs=[_row_spec, _w_spec, _v_spec, _v_spec],
    out_specs=[_row_spec, _col_spec, _col_spec],
    out_shape=[
        jax.ShapeDtypeStruct((NP, D), _f32),
        jax.ShapeDtypeStruct((NP, 1), _f32),
        jax.ShapeDtypeStruct((NP, 1), _f32),
    ],
)

_comb_mm = pl.pallas_call(
    _comb_mm_body,
    grid=(_NBLK,),
    in_specs=[_feat_spec, _den_spec, _row_spec, _col_spec, _col_spec, _v_spec,
              _w_spec, _v_spec, _v_spec],
    out_specs=[_row_spec, _col_spec, _col_spec],
    out_shape=[
        jax.ShapeDtypeStruct((NP, D), _f32),
        jax.ShapeDtypeStruct((NP, 1), _f32),
        jax.ShapeDtypeStruct((NP, 1), _f32),
    ],
)

_pool = pl.pallas_call(
    _pool_body,
    grid=(_NBLK,),
    in_specs=[_feat_spec, _den_spec, _row_spec, _col_spec, _col_spec, _v_spec,
              pl.BlockSpec((_BLK, 1), lambda i: (i, 0))],
    out_specs=pl.BlockSpec((G, D), lambda i: (0, 0)),
    out_shape=jax.ShapeDtypeStruct((G, D), _f32),
    scratch_shapes=[pltpu.VMEM((G, D), _f32), pltpu.VMEM((G, D), _f32)],
)


def kernel(node_ids, edge_index, batch, emb,
           W1, a_src1, a_dst1, b1,
           W2, a_src2, a_dst2, b2,
           W3, a_src3, a_dst3, b3):
    ids_p = jnp.pad(node_ids.astype(_i32), (0, NP - N))
    src = jnp.pad(edge_index[0].astype(_i32), (0, EP - E), constant_values=NP - 2)
    dst = jnp.pad(edge_index[1].astype(_i32), (0, EP - E), constant_values=NP - 1)
    batch_p = jnp.pad(batch.astype(_i32), (0, NP - N), constant_values=G)
    batch_p = batch_p.reshape(NP, 1)

    x = _emb_gather(ids_p, emb)

    h1, s1, d1 = _mm_alpha(x, W1, a_src1.reshape(1, D), a_dst1.reshape(1, D))
    f1, e1 = _edge_accum(h1, s1.reshape(NP), d1.reshape(NP), src, dst)
    e1 = e1.reshape(2, NP, 1)

    h2, s2, d2 = _comb_mm(f1, e1, h1, s1, d1, b1.reshape(1, D),
                          W2, a_src2.reshape(1, D), a_dst2.reshape(1, D))
    f2, e2 = _edge_accum(h2, s2.reshape(NP), d2.reshape(NP), src, dst)
    e2 = e2.reshape(2, NP, 1)

    h3, s3, d3 = _comb_mm(f2, e2, h2, s2, d2, b2.reshape(1, D),
                          W3, a_src3.reshape(1, D), a_dst3.reshape(1, D))
    f3, e3 = _edge_accum(h3, s3.reshape(NP), d3.reshape(NP), src, dst)
    e3 = e3.reshape(2, NP, 1)

    return _pool(f3, e3, h3, s3, d3, b3.reshape(1, D), batch_p)
